# Initial kernel scaffold; baseline (speedup 1.0000x reference)
import jax
import jax.numpy as jnp
from jax.experimental import pallas as pl


def kernel(times, scores, events):
    raise NotImplementedError("write your pallas kernel here")


# speedup vs baseline: 6277.4325x; 6277.4325x over previous
import functools

import jax
import jax.numpy as jnp
from jax import lax
from jax.experimental import pallas as pl
from jax.experimental.pallas import tpu as pltpu
from jax.experimental.pallas import tpu_sc as plsc

_SIGMA = 0.1
_N = 4096
_L = 16
_NC = 2
_NS = 16
_NW = _NC * _NS
_ROWS = _N // _NW
_AG = 4
_NB = _N // _L


def _bcast_lane(vec, idxv):
    return lax.gather(
        vec,
        idxv[:, None],
        lax.GatherDimensionNumbers(
            offset_dims=(), collapsed_slice_dims=(0,), start_index_map=(0,)
        ),
        (1,),
        indices_are_sorted=False,
        unique_indices=False,
        mode=lax.GatherScatterMode.PROMISE_IN_BOUNDS,
    )


def _cindex_sc_kernel(t_hbm, e_hbm, f_hbm, out_hbm, t_v, e_v, f_v, o_v):
    wid = lax.axis_index("s") * _NC + lax.axis_index("c")
    pltpu.sync_copy(t_hbm, t_v)
    pltpu.sync_copy(e_hbm, e_v)
    pltpu.sync_copy(f_hbm, f_v)

    def exp_body(i, c):
        sl = pl.ds(i * _L, _L)
        e_v[sl] = jnp.exp(e_v[sl] * (1.0 / _SIGMA))
        return c

    lax.fori_loop(0, _NB, exp_body, 0)

    base = wid * _ROWS
    zero = jnp.zeros((_L,), jnp.float32)

    def a_body(bi, carry):
        sl_a = pl.ds(base + bi * _L, _L)
        ta_blk = t_v[sl_a]
        ea_blk = e_v[sl_a]

        def k_body(kg, carry2):
            splats = []
            for j in range(_AG):
                idxv = jnp.full((_L,), kg * _AG + j, jnp.int32)
                splats.append((_bcast_lane(ta_blk, idxv), _bcast_lane(ea_blk, idxv)))

            def b_body(c, carry3):
                accn3, accd3 = carry3
                sl = pl.ds(c * _L, _L)
                tb = t_v[sl]
                eb = e_v[sl]
                fb = f_v[sl]
                for ta, ea in splats:
                    mf = jnp.where(ta > tb, fb, 0.0)
                    q = ea / (ea + eb)
                    accn3 = accn3 + q * mf
                    accd3 = accd3 + mf
                return accn3, accd3

            return lax.fori_loop(0, _NB, b_body, carry2)

        return lax.fori_loop(0, _L // _AG, k_body, carry)

    accn, accd = lax.fori_loop(0, _ROWS // _L, a_body, (zero, zero))
    o_v[pl.ds(0, _L)] = accn
    o_v[pl.ds(_L, _L)] = accd
    pltpu.sync_copy(o_v, out_hbm.at[wid])


@jax.jit
def kernel(times, scores, events):
    mesh = plsc.VectorSubcoreMesh(core_axis_name="c", subcore_axis_name="s")
    partials = pl.kernel(
        _cindex_sc_kernel,
        mesh=mesh,
        out_type=jax.ShapeDtypeStruct((_NW, 2 * _L), jnp.float32),
        scratch_types=[
            pltpu.VMEM((_N,), jnp.float32),
            pltpu.VMEM((_N,), jnp.float32),
            pltpu.VMEM((_N,), jnp.float32),
            pltpu.VMEM((2 * _L,), jnp.float32),
        ],
    )(times, scores, events.astype(jnp.float32))
    num = partials[:, :_L].sum()
    den = partials[:, _L:].sum()
    return num / (den + 1.0)
